# Initial kernel scaffold; baseline (speedup 1.0000x reference)
import jax
import jax.numpy as jnp
from jax.experimental import pallas as pl


def kernel(vectors, lengths, node_feats, node_attrs, edge_feats, edge_index, W_up, Wr1, Wr2, Wlin_s, Wlin_v, Wsp, Wprod_s, Wprod_v, W1, W2, Wv, Wgv):
    raise NotImplementedError("write your pallas kernel here")


# speedup vs baseline: 23.0774x; 23.0774x over previous
import functools
import math

import jax
import jax.numpy as jnp
from jax import lax
from jax.experimental import pallas as pl
from jax.experimental.pallas import tpu as pltpu
from jax.experimental.pallas import tpu_sc as plsc

_AVG = 32.0
_SQRT3 = math.sqrt(3.0)

_NC = 2
_NS = 16
_NW = _NC * _NS


def _pre_body(nf_ref, w_ref, o_ref):
    o_ref[...] = jnp.dot(nf_ref[...], w_ref[...],
                         preferred_element_type=jnp.float32)


def _pre_call(node_feats, W_up):
    n, d = node_feats.shape
    return pl.pallas_call(
        _pre_body,
        out_shape=jax.ShapeDtypeStruct((n, d), jnp.float32),
    )(node_feats, W_up)


def _gather_body(h_hbm, src_hbm, out_hbm, idxv, rows, sem, *, chunk, nchunk):
    c = lax.axis_index("c")
    s = lax.axis_index("s")
    wid = s * _NC + c
    per_w = chunk * nchunk
    base = wid * per_w
    pltpu.sync_copy(src_hbm.at[wid], idxv)

    def body(j, carry):
        pltpu.async_copy(h_hbm.at[idxv.at[j]], rows, sem).wait()
        pltpu.sync_copy(rows, out_hbm.at[pl.ds(base + j * chunk, chunk)])
        return carry

    lax.fori_loop(0, nchunk, body, 0)


def _gather_call(h, src3):
    nw, nchunk, chunk = src3.shape
    n, d = h.shape
    e = nw * nchunk * chunk
    mesh = plsc.VectorSubcoreMesh(core_axis_name="c", subcore_axis_name="s")
    f = pl.kernel(
        functools.partial(_gather_body, chunk=chunk, nchunk=nchunk),
        out_type=jax.ShapeDtypeStruct((e, d), jnp.float32),
        mesh=mesh,
        scratch_types=[
            pltpu.VMEM((nchunk, chunk), jnp.int32),
            pltpu.VMEM((chunk, d), jnp.float32),
            pltpu.SemaphoreType.DMA,
        ],
    )
    return f(h, src3)


def _msg_body(rad_ref, vec_ref, hs_ref, wr1_ref, wr2_ref, o_ref, *, d):
    rad = rad_ref[...]
    hid = jnp.dot(rad, wr1_ref[...], preferred_element_type=jnp.float32)
    hid = hid * jax.nn.sigmoid(hid)
    w = jnp.dot(hid, wr2_ref[...], preferred_element_type=jnp.float32)
    v = vec_ref[...]
    inv = _SQRT3 / (jnp.sqrt(jnp.sum(v * v, axis=1, keepdims=True)) + 1e-9)
    sh = v * inv
    hs = hs_ref[...]
    m0 = w[:, :d] * hs
    t = w[:, d:] * hs
    o_ref[0] = m0
    o_ref[1] = t * sh[:, 0:1]
    o_ref[2] = t * sh[:, 1:2]
    o_ref[3] = t * sh[:, 2:3]


def _msg_call(rad_in, vectors, h_s, Wr1, Wr2, blk=4000):
    e, d = h_s.shape
    nin = rad_in.shape[1]
    hidw = Wr1.shape[1]
    grid = (e // blk,)
    return pl.pallas_call(
        functools.partial(_msg_body, d=d),
        grid=grid,
        in_specs=[
            pl.BlockSpec((blk, nin), lambda i: (i, 0)),
            pl.BlockSpec((blk, 3), lambda i: (i, 0)),
            pl.BlockSpec((blk, d), lambda i: (i, 0)),
            pl.BlockSpec((nin, hidw), lambda i: (0, 0)),
            pl.BlockSpec((hidw, 2 * d), lambda i: (0, 0)),
        ],
        out_specs=pl.BlockSpec((4, blk, d), lambda i: (0, i, 0)),
        out_shape=jax.ShapeDtypeStruct((4, e, d), jnp.float32),
    )(rad_in, vectors, h_s, Wr1, Wr2)


def _scatter_body(msg_hbm, dst_hbm, zer_hbm, out_hbm, dstv, rows, acc, *,
                  chunk, nchunk, n):
    c = lax.axis_index("c")
    s = lax.axis_index("s")
    nsl = (n // _NS) // 8 * 8
    tail_base = 15 * nsl
    tail = n - tail_base
    per_s = chunk * nchunk
    pltpu.sync_copy(dst_hbm.at[s], dstv)
    for ch_local in range(2):
        ch = c * 2 + ch_local

        @pl.when(s < 15)
        def _():
            pltpu.sync_copy(zer_hbm.at[pl.ds(s * nsl, nsl)],
                            acc.at[pl.ds(s * nsl, nsl)])

        @pl.when(s == 15)
        def _():
            pltpu.sync_copy(zer_hbm.at[pl.ds(tail_base, tail)],
                            acc.at[pl.ds(tail_base, tail)])

        plsc.subcore_barrier()

        def body(j, carry):
            pltpu.sync_copy(
                msg_hbm.at[ch, pl.ds(s * per_s + j * chunk, chunk)], rows)
            pltpu.sync_copy(rows, acc.at[dstv.at[j]], add=True)
            return carry

        lax.fori_loop(0, nchunk, body, 0)
        plsc.subcore_barrier()

        @pl.when(s < 15)
        def _():
            pltpu.sync_copy(acc.at[pl.ds(s * nsl, nsl)],
                            out_hbm.at[ch, pl.ds(s * nsl, nsl)])

        @pl.when(s == 15)
        def _():
            pltpu.sync_copy(acc.at[pl.ds(tail_base, tail)],
                            out_hbm.at[ch, pl.ds(tail_base, tail)])

        plsc.subcore_barrier()


def _scatter_call(msg, dst3, zer):
    _, e, d = msg.shape
    ns, nchunk, chunk = dst3.shape
    n = zer.shape[0]
    mesh = plsc.VectorSubcoreMesh(core_axis_name="c", subcore_axis_name="s")
    f = pl.kernel(
        functools.partial(_scatter_body, chunk=chunk, nchunk=nchunk, n=n),
        out_type=jax.ShapeDtypeStruct((4, n, d), jnp.float32),
        mesh=mesh,
        scratch_types=[
            pltpu.VMEM((nchunk, chunk), jnp.int32),
            pltpu.VMEM((chunk, d), jnp.float32),
            pltpu.VMEM_SHARED((n, d), jnp.float32),
        ],
    )
    return f(msg, dst3, zer)


def _post_body(agg_ref, na_ref, wlin_s, wlin_v, wsp, wprod_s, wprod_v,
               w1, w2, wv, wgv, os_ref, vec_ref, s2_ref, v2x_ref, v2y_ref,
               v2z_ref):
    f32 = jnp.float32
    agg = agg_ref[...] * (1.0 / _AVG)
    s = jnp.dot(agg[0], wlin_s[...], preferred_element_type=f32)
    vx = jnp.dot(agg[1], wlin_v[...], preferred_element_type=f32)
    vy = jnp.dot(agg[2], wlin_v[...], preferred_element_type=f32)
    vz = jnp.dot(agg[3], wlin_v[...], preferred_element_type=f32)
    na = na_ref[...]
    wsp_all = wsp[...]
    c1 = jnp.dot(na, wsp_all[0], preferred_element_type=f32)
    c2 = jnp.dot(na, wsp_all[1], preferred_element_type=f32)
    c3 = jnp.dot(na, wsp_all[2], preferred_element_type=f32)
    vnorm2 = vx * vx + vy * vy + vz * vz
    ss = s * s
    s_out = c1 * s + c2 * (ss + vnorm2) + c3 * (ss * s + s * vnorm2)
    g = c1 + c2 * s + c3 * ss
    s2 = jnp.dot(s_out, wprod_s[...], preferred_element_type=f32)
    v2x = jnp.dot(g * vx, wprod_v[...], preferred_element_type=f32)
    v2y = jnp.dot(g * vy, wprod_v[...], preferred_element_type=f32)
    v2z = jnp.dot(g * vz, wprod_v[...], preferred_element_type=f32)
    hid = jnp.dot(s2, w1[...], preferred_element_type=f32)
    hid = hid * jax.nn.sigmoid(hid)
    out_s = jnp.dot(hid, w2[...], preferred_element_type=f32)
    gpre = jnp.dot(s2, wgv[...], preferred_element_type=f32)
    gate = gpre * jax.nn.sigmoid(gpre)
    wv_all = wv[...]
    vec = jnp.concatenate(
        [jnp.dot(v2x, wv_all, preferred_element_type=f32),
         jnp.dot(v2y, wv_all, preferred_element_type=f32),
         jnp.dot(v2z, wv_all, preferred_element_type=f32)], axis=1) * gate
    os_ref[...] = out_s
    vec_ref[...] = vec
    s2_ref[...] = s2
    v2x_ref[...] = v2x
    v2y_ref[...] = v2y
    v2z_ref[...] = v2z


def _post_call(agg, node_attrs, Wlin_s, Wlin_v, Wsp, Wprod_s, Wprod_v,
               W1, W2, Wv, Wgv, blk=2000):
    _, n, d = agg.shape
    ns_ = node_attrs.shape[1]
    mlph = W1.shape[1]
    grid = (n // blk,)
    wspec = lambda shape: pl.BlockSpec(shape, lambda i: tuple(0 for _ in shape))
    out_shapes = (
        jax.ShapeDtypeStruct((n, d), jnp.float32),
        jax.ShapeDtypeStruct((n, 3), jnp.float32),
        jax.ShapeDtypeStruct((n, d), jnp.float32),
        jax.ShapeDtypeStruct((n, d), jnp.float32),
        jax.ShapeDtypeStruct((n, d), jnp.float32),
        jax.ShapeDtypeStruct((n, d), jnp.float32),
    )
    return pl.pallas_call(
        _post_body,
        grid=grid,
        in_specs=[
            pl.BlockSpec((4, blk, d), lambda i: (0, i, 0)),
            pl.BlockSpec((blk, ns_), lambda i: (i, 0)),
            wspec((d, d)),
            wspec((d, d)),
            wspec((3, ns_, d)),
            wspec((d, d)),
            wspec((d, d)),
            wspec((d, mlph)),
            wspec((mlph, d)),
            wspec((d, 1)),
            wspec((d, 1)),
        ],
        out_specs=(
            pl.BlockSpec((blk, d), lambda i: (i, 0)),
            pl.BlockSpec((blk, 3), lambda i: (i, 0)),
            pl.BlockSpec((blk, d), lambda i: (i, 0)),
            pl.BlockSpec((blk, d), lambda i: (i, 0)),
            pl.BlockSpec((blk, d), lambda i: (i, 0)),
            pl.BlockSpec((blk, d), lambda i: (i, 0)),
        ),
        out_shape=out_shapes,
    )(agg, node_attrs, Wlin_s, Wlin_v, Wsp, Wprod_s, Wprod_v, W1, W2, Wv,
      Wgv)


def kernel(vectors, lengths, node_feats, node_attrs, edge_feats, edge_index,
           W_up, Wr1, Wr2, Wlin_s, Wlin_v, Wsp, Wprod_s, Wprod_v, W1, W2,
           Wv, Wgv):
    n, d = node_feats.shape
    e = vectors.shape[0]
    src = edge_index[0]
    dst = edge_index[1]

    h = _pre_call(node_feats, W_up)

    chunk = 80
    per_w = e // _NW
    src3 = src.reshape(_NW, per_w // chunk, chunk)
    h_s = _gather_call(h, src3)

    rad_in = jnp.concatenate([edge_feats, lengths], axis=1)
    msg = _msg_call(rad_in, vectors, h_s, Wr1, Wr2)

    per_s = e // _NS
    dst3 = dst.reshape(_NS, per_s // chunk, chunk)
    zer = jnp.zeros((n, d), jnp.float32)
    agg = _scatter_call(msg, dst3, zer)

    out_s, vec, s2, v2x, v2y, v2z = _post_call(
        agg, node_attrs, Wlin_s, Wlin_v, Wsp, Wprod_s, Wprod_v, W1, W2,
        Wv, Wgv)

    v2 = jnp.stack([v2x, v2y, v2z], axis=-1)
    node_feats_out = jnp.concatenate([s2, v2.reshape(n, 3 * d)], axis=1)
    return (out_s, vec, node_feats_out)


# speedup vs baseline: 29.5969x; 1.2825x over previous
import functools
import math

import jax
import jax.numpy as jnp
from jax import lax
from jax.experimental import pallas as pl
from jax.experimental.pallas import tpu as pltpu
from jax.experimental.pallas import tpu_sc as plsc

_AVG = 32.0
_SQRT3 = math.sqrt(3.0)

_NC = 2
_NS = 16
_NW = _NC * _NS


def _pre_body(nf_ref, w_ref, o_ref):
    o_ref[...] = jnp.dot(nf_ref[...], w_ref[...],
                         preferred_element_type=jnp.float32)


def _pre_call(node_feats, W_up):
    n, d = node_feats.shape
    return pl.pallas_call(
        _pre_body,
        out_shape=jax.ShapeDtypeStruct((n, d), jnp.float32),
    )(node_feats, W_up)


def _gather_body(h_hbm, src_hbm, out_hbm, idxv, rows0, rows1, sem0, sem1,
                 *, chunk, nchunk):
    c = lax.axis_index("c")
    s = lax.axis_index("s")
    wid = s * _NC + c
    per_w = chunk * nchunk
    base = wid * per_w
    rows = (rows0, rows1)
    sem = (sem0, sem1)
    pltpu.sync_copy(src_hbm.at[wid], idxv)
    pltpu.async_copy(h_hbm.at[idxv.at[0]], rows0, sem0)

    def outer(p, carry):
        for b in range(2):
            j = p * 2 + b

            @pl.when(j < nchunk)
            def _():
                @pl.when(j + 1 < nchunk)
                def _():
                    pltpu.async_copy(h_hbm.at[idxv.at[j + 1]], rows[1 - b],
                                     sem[1 - b])

                pltpu.make_async_copy(h_hbm.at[idxv.at[j]], rows[b],
                                      sem[b]).wait()
                pltpu.sync_copy(rows[b],
                                out_hbm.at[pl.ds(base + j * chunk, chunk)])

        return carry

    lax.fori_loop(0, (nchunk + 1) // 2, outer, 0)


def _gather_call(h, src3):
    nw, nchunk, chunk = src3.shape
    n, d = h.shape
    e = nw * nchunk * chunk
    mesh = plsc.VectorSubcoreMesh(core_axis_name="c", subcore_axis_name="s")
    f = pl.kernel(
        functools.partial(_gather_body, chunk=chunk, nchunk=nchunk),
        out_type=jax.ShapeDtypeStruct((e, d), jnp.float32),
        mesh=mesh,
        scratch_types=[
            pltpu.VMEM((nchunk, chunk), jnp.int32),
            pltpu.VMEM((chunk, d), jnp.float32),
            pltpu.VMEM((chunk, d), jnp.float32),
            pltpu.SemaphoreType.DMA,
            pltpu.SemaphoreType.DMA,
        ],
    )
    return f(h, src3)


def _msg_body(rad_ref, vec_ref, hs_ref, wr1_ref, wr2_ref, o_ref, *, d):
    rad = rad_ref[...]
    hid = jnp.dot(rad, wr1_ref[...], preferred_element_type=jnp.float32)
    hid = hid * jax.nn.sigmoid(hid)
    w = jnp.dot(hid, wr2_ref[...], preferred_element_type=jnp.float32)
    v = vec_ref[...]
    inv = _SQRT3 / (jnp.sqrt(jnp.sum(v * v, axis=1, keepdims=True)) + 1e-9)
    sh = v * inv
    hs = hs_ref[...]
    m0 = w[:, :d] * hs
    t = w[:, d:] * hs
    o_ref[0] = m0
    o_ref[1] = t * sh[:, 0:1]
    o_ref[2] = t * sh[:, 1:2]
    o_ref[3] = t * sh[:, 2:3]


def _msg_call(rad_in, vectors, h_s, Wr1, Wr2, blk=4000):
    e, d = h_s.shape
    nin = rad_in.shape[1]
    hidw = Wr1.shape[1]
    grid = (e // blk,)
    return pl.pallas_call(
        functools.partial(_msg_body, d=d),
        grid=grid,
        in_specs=[
            pl.BlockSpec((blk, nin), lambda i: (i, 0)),
            pl.BlockSpec((blk, 3), lambda i: (i, 0)),
            pl.BlockSpec((blk, d), lambda i: (i, 0)),
            pl.BlockSpec((nin, hidw), lambda i: (0, 0)),
            pl.BlockSpec((hidw, 2 * d), lambda i: (0, 0)),
        ],
        out_specs=pl.BlockSpec((4, blk, d), lambda i: (0, i, 0)),
        out_shape=jax.ShapeDtypeStruct((4, e, d), jnp.float32),
    )(rad_in, vectors, h_s, Wr1, Wr2)


def _scatter_body(msg_hbm, dst_hbm, zer_hbm, out_hbm, idx0, idx1, rows0,
                  rows1, semi0, semi1, semr0, semr1, acc, *, chunk, nchunk,
                  g, n):
    c = lax.axis_index("c")
    s = lax.axis_index("s")
    nsl = (n // _NS) // 8 * 8
    tail_base = 15 * nsl
    tail = n - tail_base
    per_s = chunk * nchunk
    ngroups = g
    gsz = nchunk // g
    idxg = (idx0, idx1)
    rows = (rows0, rows1)
    semi = (semi0, semi1)
    semr = (semr0, semr1)
    for ch_local in range(2):
        ch = c * 2 + ch_local

        @pl.when(s < 15)
        def _():
            pltpu.sync_copy(zer_hbm.at[pl.ds(s * nsl, nsl)],
                            acc.at[pl.ds(s * nsl, nsl)])

        @pl.when(s == 15)
        def _():
            pltpu.sync_copy(zer_hbm.at[pl.ds(tail_base, tail)],
                            acc.at[pl.ds(tail_base, tail)])

        plsc.subcore_barrier()

        pltpu.async_copy(dst_hbm.at[s * ngroups], idx0, semi0)
        pltpu.async_copy(msg_hbm.at[ch, pl.ds(s * per_s, chunk)], rows0,
                         semr0)

        def outer(p, carry):
            for b in range(2):
                gi = p * 2 + b

                @pl.when(gi < ngroups)
                def _():
                    @pl.when(gi + 1 < ngroups)
                    def _():
                        pltpu.async_copy(dst_hbm.at[s * ngroups + gi + 1],
                                         idxg[1 - b], semi[1 - b])

                    pltpu.make_async_copy(dst_hbm.at[s * ngroups + gi],
                                          idxg[b], semi[b]).wait()

                    def inner(q, icarry):
                        for rr in range(2):
                            r = q * 2 + rr
                            j = gi * gsz + r

                            @pl.when(j + 1 < nchunk)
                            def _():
                                pltpu.async_copy(
                                    msg_hbm.at[ch,
                                               pl.ds(s * per_s +
                                                     (j + 1) * chunk,
                                                     chunk)],
                                    rows[1 - rr], semr[1 - rr])

                            pltpu.make_async_copy(
                                msg_hbm.at[ch, pl.ds(s * per_s + j * chunk,
                                                     chunk)],
                                rows[rr], semr[rr]).wait()
                            pltpu.sync_copy(rows[rr],
                                            acc.at[idxg[b].at[r]],
                                            add=True)
                        return icarry

                    lax.fori_loop(0, gsz // 2, inner, 0)

            return carry

        lax.fori_loop(0, (ngroups + 1) // 2, outer, 0)
        plsc.subcore_barrier()

        @pl.when(s < 15)
        def _():
            pltpu.sync_copy(acc.at[pl.ds(s * nsl, nsl)],
                            out_hbm.at[ch, pl.ds(s * nsl, nsl)])

        @pl.when(s == 15)
        def _():
            pltpu.sync_copy(acc.at[pl.ds(tail_base, tail)],
                            out_hbm.at[ch, pl.ds(tail_base, tail)])

        plsc.subcore_barrier()


def _scatter_call(msg, dst4, zer):
    _, e, d = msg.shape
    nrow, gsz, chunk = dst4.shape
    g = nrow // _NS
    nchunk = g * gsz
    n = zer.shape[0]
    mesh = plsc.VectorSubcoreMesh(core_axis_name="c", subcore_axis_name="s")
    f = pl.kernel(
        functools.partial(_scatter_body, chunk=chunk, nchunk=nchunk, g=g,
                          n=n),
        out_type=jax.ShapeDtypeStruct((4, n, d), jnp.float32),
        mesh=mesh,
        scratch_types=[
            pltpu.VMEM((gsz, chunk), jnp.int32),
            pltpu.VMEM((gsz, chunk), jnp.int32),
            pltpu.VMEM((chunk, d), jnp.float32),
            pltpu.VMEM((chunk, d), jnp.float32),
            pltpu.SemaphoreType.DMA,
            pltpu.SemaphoreType.DMA,
            pltpu.SemaphoreType.DMA,
            pltpu.SemaphoreType.DMA,
            pltpu.VMEM_SHARED((n, d), jnp.float32),
        ],
    )
    return f(msg, dst4, zer)


def _post_body(agg_ref, na_ref, wlin_s, wlin_v, wsp, wprod_s, wprod_v,
               w1, w2, wv, wgv, os_ref, vec_ref, s2_ref, v2x_ref, v2y_ref,
               v2z_ref):
    f32 = jnp.float32
    agg = agg_ref[...] * (1.0 / _AVG)
    s = jnp.dot(agg[0], wlin_s[...], preferred_element_type=f32)
    vx = jnp.dot(agg[1], wlin_v[...], preferred_element_type=f32)
    vy = jnp.dot(agg[2], wlin_v[...], preferred_element_type=f32)
    vz = jnp.dot(agg[3], wlin_v[...], preferred_element_type=f32)
    na = na_ref[...]
    wsp_all = wsp[...]
    c1 = jnp.dot(na, wsp_all[0], preferred_element_type=f32)
    c2 = jnp.dot(na, wsp_all[1], preferred_element_type=f32)
    c3 = jnp.dot(na, wsp_all[2], preferred_element_type=f32)
    vnorm2 = vx * vx + vy * vy + vz * vz
    ss = s * s
    s_out = c1 * s + c2 * (ss + vnorm2) + c3 * (ss * s + s * vnorm2)
    g = c1 + c2 * s + c3 * ss
    s2 = jnp.dot(s_out, wprod_s[...], preferred_element_type=f32)
    v2x = jnp.dot(g * vx, wprod_v[...], preferred_element_type=f32)
    v2y = jnp.dot(g * vy, wprod_v[...], preferred_element_type=f32)
    v2z = jnp.dot(g * vz, wprod_v[...], preferred_element_type=f32)
    hid = jnp.dot(s2, w1[...], preferred_element_type=f32)
    hid = hid * jax.nn.sigmoid(hid)
    out_s = jnp.dot(hid, w2[...], preferred_element_type=f32)
    gpre = jnp.dot(s2, wgv[...], preferred_element_type=f32)
    gate = gpre * jax.nn.sigmoid(gpre)
    wv_all = wv[...]
    vec = jnp.concatenate(
        [jnp.dot(v2x, wv_all, preferred_element_type=f32),
         jnp.dot(v2y, wv_all, preferred_element_type=f32),
         jnp.dot(v2z, wv_all, preferred_element_type=f32)], axis=1) * gate
    os_ref[...] = out_s
    vec_ref[...] = vec
    s2_ref[...] = s2
    v2x_ref[...] = v2x
    v2y_ref[...] = v2y
    v2z_ref[...] = v2z


def _post_call(agg, node_attrs, Wlin_s, Wlin_v, Wsp, Wprod_s, Wprod_v,
               W1, W2, Wv, Wgv, blk=2000):
    _, n, d = agg.shape
    ns_ = node_attrs.shape[1]
    mlph = W1.shape[1]
    grid = (n // blk,)
    wspec = lambda shape: pl.BlockSpec(shape, lambda i: tuple(0 for _ in shape))
    out_shapes = (
        jax.ShapeDtypeStruct((n, d), jnp.float32),
        jax.ShapeDtypeStruct((n, 3), jnp.float32),
        jax.ShapeDtypeStruct((n, d), jnp.float32),
        jax.ShapeDtypeStruct((n, d), jnp.float32),
        jax.ShapeDtypeStruct((n, d), jnp.float32),
        jax.ShapeDtypeStruct((n, d), jnp.float32),
    )
    return pl.pallas_call(
        _post_body,
        grid=grid,
        in_specs=[
            pl.BlockSpec((4, blk, d), lambda i: (0, i, 0)),
            pl.BlockSpec((blk, ns_), lambda i: (i, 0)),
            wspec((d, d)),
            wspec((d, d)),
            wspec((3, ns_, d)),
            wspec((d, d)),
            wspec((d, d)),
            wspec((d, mlph)),
            wspec((mlph, d)),
            wspec((d, 1)),
            wspec((d, 1)),
        ],
        out_specs=(
            pl.BlockSpec((blk, d), lambda i: (i, 0)),
            pl.BlockSpec((blk, 3), lambda i: (i, 0)),
            pl.BlockSpec((blk, d), lambda i: (i, 0)),
            pl.BlockSpec((blk, d), lambda i: (i, 0)),
            pl.BlockSpec((blk, d), lambda i: (i, 0)),
            pl.BlockSpec((blk, d), lambda i: (i, 0)),
        ),
        out_shape=out_shapes,
    )(agg, node_attrs, Wlin_s, Wlin_v, Wsp, Wprod_s, Wprod_v, W1, W2, Wv,
      Wgv)


def kernel(vectors, lengths, node_feats, node_attrs, edge_feats, edge_index,
           W_up, Wr1, Wr2, Wlin_s, Wlin_v, Wsp, Wprod_s, Wprod_v, W1, W2,
           Wv, Wgv):
    n, d = node_feats.shape
    e = vectors.shape[0]
    src = edge_index[0]
    dst = edge_index[1]

    h = _pre_call(node_feats, W_up)

    chunk = 80
    per_w = e // _NW
    src3 = src.reshape(_NW, per_w // chunk, chunk)
    h_s = _gather_call(h, src3)

    rad_in = jnp.concatenate([edge_feats, lengths], axis=1)
    msg = _msg_call(rad_in, vectors, h_s, Wr1, Wr2)

    g = 5
    per_s = e // _NS
    gsz = per_s // chunk // g
    dst4 = dst.reshape(_NS * g, gsz, chunk)
    zer = jnp.zeros((n, d), jnp.float32)
    agg = _scatter_call(msg, dst4, zer)

    out_s, vec, s2, v2x, v2y, v2z = _post_call(
        agg, node_attrs, Wlin_s, Wlin_v, Wsp, Wprod_s, Wprod_v, W1, W2,
        Wv, Wgv)

    v2 = jnp.stack([v2x, v2y, v2z], axis=-1)
    node_feats_out = jnp.concatenate([s2, v2.reshape(n, 3 * d)], axis=1)
    return (out_s, vec, node_feats_out)
